# baseline (speedup 1.0000x reference)
import functools

import jax
import jax.numpy as jnp
from jax import lax
from jax.experimental import pallas as pl
from jax.experimental.pallas import tpu as pltpu
from jax.experimental.pallas import tpu_sc as plsc

S, D, F, E = 2048, 1024, 4096, 8
BS_G = 256
NB = 2 * S // BS_G + E - 1
P = NB * BS_G
FT = 2048

NC, NS = 2, 16
NW = NC * NS
CHUNK = S // NW


def _router_kernel(x_ref, rw_ref, rb_ref, p0_ref, p1_ref, w0_ref, w1_ref,
                   bexp_ref, xbf_ref):
    xbf_ref[...] = x_ref[...].astype(jnp.bfloat16)
    logits = jnp.dot(x_ref[...], rw_ref[...],
                     preferred_element_type=jnp.float32) + rb_ref[...]
    m = jnp.max(logits, axis=1, keepdims=True)
    p = jnp.exp(logits - m)
    probs = p / jnp.sum(p, axis=1, keepdims=True)
    lane = lax.broadcasted_iota(jnp.int32, (S, E), 1)
    m0 = jnp.max(probs, axis=1, keepdims=True)
    idx0 = jnp.min(jnp.where(probs == m0, lane, E), axis=1, keepdims=True)
    oh0 = (lane == idx0).astype(jnp.float32)
    probs2 = jnp.where(lane == idx0, -1.0, probs)
    m1 = jnp.max(probs2, axis=1, keepdims=True)
    idx1 = jnp.min(jnp.where(probs2 == m1, lane, E), axis=1, keepdims=True)
    oh1 = (lane == idx1).astype(jnp.float32)
    denom = m0 + m1 + 1e-9
    w0_ref[...] = m0 / denom
    w1_ref[...] = m1 / denom

    TB = 256
    r = lax.broadcasted_iota(jnp.int32, (TB, TB), 0)
    c = lax.broadcasted_iota(jnp.int32, (TB, TB), 1)
    L = (c < r).astype(jnp.float32)

    def excl_ranks(oh):
        parts = []
        prev = jnp.zeros((1, E), dtype=jnp.float32)
        for g in range(S // TB):
            blk = oh[g * TB:(g + 1) * TB, :]
            parts.append(jnp.dot(L, blk, preferred_element_type=jnp.float32)
                         + prev)
            prev = prev + jnp.sum(blk, axis=0, keepdims=True)
        return jnp.concatenate(parts, axis=0), prev

    c0x, cnt0 = excl_ranks(oh0)
    c1x, cnt1 = excl_ranks(oh1)
    count = cnt0 + cnt1
    nblk = jnp.floor((count + (BS_G - 1)) * (1.0 / BS_G))
    e_r = lax.broadcasted_iota(jnp.int32, (E, E), 0)
    e_c = lax.broadcasted_iota(jnp.int32, (E, E), 1)
    M8 = (e_r < e_c).astype(jnp.float32)
    blockstart = jnp.dot(nblk, M8, preferred_element_type=jnp.float32)
    off = blockstart * float(BS_G)
    p0 = jnp.sum(oh0 * (off + c0x), axis=1, keepdims=True)
    p1 = jnp.sum(oh1 * (off + cnt0 + c1x), axis=1, keepdims=True)
    p0_ref[...] = p0.astype(jnp.int32)
    p1_ref[...] = p1.astype(jnp.int32)

    b_i = lax.broadcasted_iota(jnp.int32, (NB, E), 0).astype(jnp.float32)
    ge = (b_i >= blockstart).astype(jnp.float32)
    bexp_ref[...] = (jnp.sum(ge, axis=1, keepdims=True) - 1.0).astype(jnp.int32)


def _router(x, rw, rb):
    return pl.pallas_call(
        _router_kernel,
        grid=(1,),
        in_specs=[
            pl.BlockSpec((S, D), lambda i: (0, 0)),
            pl.BlockSpec((D, E), lambda i: (0, 0)),
            pl.BlockSpec((1, E), lambda i: (0, 0)),
        ],
        out_specs=[
            pl.BlockSpec((S, 1), lambda i: (0, 0)),
            pl.BlockSpec((S, 1), lambda i: (0, 0)),
            pl.BlockSpec((S, 1), lambda i: (0, 0)),
            pl.BlockSpec((S, 1), lambda i: (0, 0)),
            pl.BlockSpec((NB, 1), lambda i: (0, 0)),
            pl.BlockSpec((S, D), lambda i: (0, 0)),
        ],
        out_shape=[
            jax.ShapeDtypeStruct((S, 1), jnp.int32),
            jax.ShapeDtypeStruct((S, 1), jnp.int32),
            jax.ShapeDtypeStruct((S, 1), jnp.float32),
            jax.ShapeDtypeStruct((S, 1), jnp.float32),
            jax.ShapeDtypeStruct((NB, 1), jnp.int32),
            jax.ShapeDtypeStruct((S, D), jnp.bfloat16),
        ],
    )(x, rw, rb.reshape(1, E))


def _sc_mesh():
    return plsc.VectorSubcoreMesh(core_axis_name="c", subcore_axis_name="s")


def _scatter_rows(x, p0, p1):
    @functools.partial(
        pl.kernel, mesh=_sc_mesh(),
        out_type=jax.ShapeDtypeStruct((P, D // 2), jnp.int32),
        scratch_types=[
            pltpu.VMEM((CHUNK,), jnp.int32),
            pltpu.VMEM((CHUNK,), jnp.int32),
            pltpu.VMEM((CHUNK, D // 2), jnp.int32),
        ],
    )
    def k(x_hbm, p0_hbm, p1_hbm, xs_hbm, i0_v, i1_v, rows_v):
        wid = lax.axis_index("s") * NC + lax.axis_index("c")
        base = wid * CHUNK
        pltpu.sync_copy(p0_hbm.at[pl.ds(base, CHUNK)], i0_v)
        pltpu.sync_copy(p1_hbm.at[pl.ds(base, CHUNK)], i1_v)
        pltpu.sync_copy(x_hbm.at[pl.ds(base, CHUNK)], rows_v)
        pltpu.sync_copy(rows_v, xs_hbm.at[i0_v])
        pltpu.sync_copy(rows_v, xs_hbm.at[i1_v])

    return k(x, p0, p1)


def _ffn_kernel(bexp_ref, xs_ref, w1_ref, b1_ref, w2_ref, b2_ref, ys_ref):
    del bexp_ref
    xb = xs_ref[...]
    y = jnp.zeros((BS_G, D), dtype=jnp.float32)
    for ft in range(F // FT):
        w1t = w1_ref[0, :, ft * FT:(ft + 1) * FT]
        b1t = b1_ref[0, 0, ft * FT:(ft + 1) * FT]
        h = jnp.dot(xb, w1t, preferred_element_type=jnp.float32)
        h = h + b1t[None, :]
        h = 0.5 * h * (1.0 + lax.erf(h * 0.7071067811865476))
        w2t = w2_ref[0, ft * FT:(ft + 1) * FT, :]
        y = y + jnp.dot(h.astype(jnp.bfloat16), w2t,
                        preferred_element_type=jnp.float32)
    ys_ref[...] = y + b2_ref[0, 0, :][None, :]


def _grouped_ffn(xs, bexp, W1b, b1, W2b, b2):
    return pl.pallas_call(
        _ffn_kernel,
        grid_spec=pltpu.PrefetchScalarGridSpec(
            num_scalar_prefetch=1,
            grid=(NB,),
            in_specs=[
                pl.BlockSpec((BS_G, D), lambda i, b: (i, 0)),
                pl.BlockSpec((1, D, F), lambda i, b: (b[i], 0, 0)),
                pl.BlockSpec((1, 1, F), lambda i, b: (b[i], 0, 0)),
                pl.BlockSpec((1, F, D), lambda i, b: (b[i], 0, 0)),
                pl.BlockSpec((1, 1, D), lambda i, b: (b[i], 0, 0)),
            ],
            out_specs=pl.BlockSpec((BS_G, D), lambda i, b: (i, 0)),
        ),
        out_shape=jax.ShapeDtypeStruct((P, D), jnp.float32),
        compiler_params=pltpu.CompilerParams(
            dimension_semantics=("parallel",)),
    )(bexp, xs, W1b, b1.reshape(E, 1, F), W2b, b2.reshape(E, 1, D))


def _gather_rows(ys, p0, p1):
    @functools.partial(
        pl.kernel, mesh=_sc_mesh(),
        out_type=[jax.ShapeDtypeStruct((S, D), jnp.float32),
                  jax.ShapeDtypeStruct((S, D), jnp.float32)],
        scratch_types=[
            pltpu.VMEM((CHUNK,), jnp.int32),
            pltpu.VMEM((CHUNK, D), jnp.float32),
        ],
    )
    def k(ys_hbm, p0_hbm, p1_hbm, y0_hbm, y1_hbm, i_v, rows_v):
        wid = lax.axis_index("s") * NC + lax.axis_index("c")
        base = wid * CHUNK
        pltpu.sync_copy(p0_hbm.at[pl.ds(base, CHUNK)], i_v)
        pltpu.sync_copy(ys_hbm.at[i_v], rows_v)
        pltpu.sync_copy(rows_v, y0_hbm.at[pl.ds(base, CHUNK)])
        pltpu.sync_copy(p1_hbm.at[pl.ds(base, CHUNK)], i_v)
        pltpu.sync_copy(ys_hbm.at[i_v], rows_v)
        pltpu.sync_copy(rows_v, y1_hbm.at[pl.ds(base, CHUNK)])

    return k(ys, p0, p1)


def _combine_kernel(y0_ref, y1_ref, w0_ref, w1_ref, out_ref):
    out_ref[...] = w0_ref[...] * y0_ref[...] + w1_ref[...] * y1_ref[...]


def _combine(y0, y1, w0, w1):
    TSC = 512
    return pl.pallas_call(
        _combine_kernel,
        grid=(S // TSC,),
        in_specs=[
            pl.BlockSpec((TSC, D), lambda i: (i, 0)),
            pl.BlockSpec((TSC, D), lambda i: (i, 0)),
            pl.BlockSpec((TSC, 1), lambda i: (i, 0)),
            pl.BlockSpec((TSC, 1), lambda i: (i, 0)),
        ],
        out_specs=pl.BlockSpec((TSC, D), lambda i: (i, 0)),
        out_shape=jax.ShapeDtypeStruct((S, D), jnp.float32),
    )(y0, y1, w0, w1)


@jax.jit
def _moe(x, rw, rb, W1, b1, W2, b2):
    xf = x.reshape(S, D)
    p0, p1, w0, w1, bexp, xbf = _router(xf, rw, rb)
    p0f, p1f = p0.reshape(S), p1.reshape(S)
    xi = lax.bitcast_convert_type(xbf.reshape(S, D // 2, 2), jnp.int32)
    xs_i = _scatter_rows(xi, p0f, p1f)
    xs = lax.bitcast_convert_type(xs_i, jnp.bfloat16).reshape(P, D)
    ys = _grouped_ffn(xs, bexp.reshape(NB),
                      W1.astype(jnp.bfloat16), b1,
                      W2.astype(jnp.bfloat16), b2)
    y0, y1 = _gather_rows(ys, p0f, p1f)
    return _combine(y0, y1, w0, w1)


def kernel(x, rw, rb, W1, b1, W2, b2):
    leading = x.shape[:-1]
    out = _moe(x, rw, rb, W1, b1, W2, b2)
    return out.reshape(*leading, D)


# speedup vs baseline: 1.5340x; 1.5340x over previous
import functools

import jax
import jax.numpy as jnp
from jax import lax
from jax.experimental import pallas as pl
from jax.experimental.pallas import tpu as pltpu
from jax.experimental.pallas import tpu_sc as plsc

S, D, F, E = 2048, 1024, 4096, 8
BS_G = 256
NB = 2 * S // BS_G + E - 1
P = NB * BS_G
FT = 2048

NC, NS = 2, 16
NW = NC * NS
CHUNK = S // NW


def _router_kernel(x_ref, rw_ref, rb_ref, p0_ref, p1_ref, w0_ref, w1_ref,
                   bexp_ref):
    logits = jnp.dot(x_ref[...], rw_ref[...],
                     preferred_element_type=jnp.float32) + rb_ref[...]
    m = jnp.max(logits, axis=1, keepdims=True)
    p = jnp.exp(logits - m)
    probs = p / jnp.sum(p, axis=1, keepdims=True)
    lane = lax.broadcasted_iota(jnp.int32, (S, E), 1)
    m0 = jnp.max(probs, axis=1, keepdims=True)
    idx0 = jnp.min(jnp.where(probs == m0, lane, E), axis=1, keepdims=True)
    oh0 = (lane == idx0).astype(jnp.float32)
    probs2 = jnp.where(lane == idx0, -1.0, probs)
    m1 = jnp.max(probs2, axis=1, keepdims=True)
    idx1 = jnp.min(jnp.where(probs2 == m1, lane, E), axis=1, keepdims=True)
    oh1 = (lane == idx1).astype(jnp.float32)
    denom = m0 + m1 + 1e-9
    w0_ref[...] = m0 / denom
    w1_ref[...] = m1 / denom

    TB = 256
    r = lax.broadcasted_iota(jnp.int32, (TB, TB), 0)
    c = lax.broadcasted_iota(jnp.int32, (TB, TB), 1)
    L = (c < r).astype(jnp.float32)

    def excl_ranks(oh):
        parts = []
        prev = jnp.zeros((1, E), dtype=jnp.float32)
        for g in range(S // TB):
            blk = oh[g * TB:(g + 1) * TB, :]
            parts.append(jnp.dot(L, blk, preferred_element_type=jnp.float32)
                         + prev)
            prev = prev + jnp.sum(blk, axis=0, keepdims=True)
        return jnp.concatenate(parts, axis=0), prev

    c0x, cnt0 = excl_ranks(oh0)
    c1x, cnt1 = excl_ranks(oh1)
    count = cnt0 + cnt1
    nblk = jnp.floor((count + (BS_G - 1)) * (1.0 / BS_G))
    e_r = lax.broadcasted_iota(jnp.int32, (E, E), 0)
    e_c = lax.broadcasted_iota(jnp.int32, (E, E), 1)
    M8 = (e_r < e_c).astype(jnp.float32)
    blockstart = jnp.dot(nblk, M8, preferred_element_type=jnp.float32)
    off = blockstart * float(BS_G)
    p0 = jnp.sum(oh0 * (off + c0x), axis=1, keepdims=True)
    p1 = jnp.sum(oh1 * (off + cnt0 + c1x), axis=1, keepdims=True)
    p0_ref[...] = p0.astype(jnp.int32)
    p1_ref[...] = p1.astype(jnp.int32)

    b_i = lax.broadcasted_iota(jnp.int32, (NB, E), 0).astype(jnp.float32)
    ge = (b_i >= blockstart).astype(jnp.float32)
    bexp_ref[...] = (jnp.sum(ge, axis=1, keepdims=True) - 1.0).astype(jnp.int32)


def _router(x, rw, rb):
    return pl.pallas_call(
        _router_kernel,
        grid=(1,),
        in_specs=[
            pl.BlockSpec((S, D), lambda i: (0, 0)),
            pl.BlockSpec((D, E), lambda i: (0, 0)),
            pl.BlockSpec((1, E), lambda i: (0, 0)),
        ],
        out_specs=[
            pl.BlockSpec((S, 1), lambda i: (0, 0)),
            pl.BlockSpec((S, 1), lambda i: (0, 0)),
            pl.BlockSpec((S, 1), lambda i: (0, 0)),
            pl.BlockSpec((S, 1), lambda i: (0, 0)),
            pl.BlockSpec((NB, 1), lambda i: (0, 0)),
        ],
        out_shape=[
            jax.ShapeDtypeStruct((S, 1), jnp.int32),
            jax.ShapeDtypeStruct((S, 1), jnp.int32),
            jax.ShapeDtypeStruct((S, 1), jnp.float32),
            jax.ShapeDtypeStruct((S, 1), jnp.float32),
            jax.ShapeDtypeStruct((NB, 1), jnp.int32),
        ],
    )(x, rw, rb.reshape(1, E))


def _sc_mesh():
    return plsc.VectorSubcoreMesh(core_axis_name="c", subcore_axis_name="s")


def _scatter_rows(x, p0, p1):
    @functools.partial(
        pl.kernel, mesh=_sc_mesh(),
        out_type=jax.ShapeDtypeStruct((P, D), jnp.float32),
        scratch_types=[
            pltpu.VMEM((CHUNK,), jnp.int32),
            pltpu.VMEM((CHUNK,), jnp.int32),
            pltpu.VMEM((CHUNK, D), jnp.float32),
            pltpu.SemaphoreType.DMA,
            pltpu.SemaphoreType.DMA,
            pltpu.SemaphoreType.DMA,
        ],
    )
    def k(x_hbm, p0_hbm, p1_hbm, xs_hbm, i0_v, i1_v, rows_v, s0, s1, s2):
        wid = lax.axis_index("s") * NC + lax.axis_index("c")
        base = wid * CHUNK
        c0 = pltpu.async_copy(p0_hbm.at[pl.ds(base, CHUNK)], i0_v, s0)
        c1 = pltpu.async_copy(p1_hbm.at[pl.ds(base, CHUNK)], i1_v, s1)
        c2 = pltpu.async_copy(x_hbm.at[pl.ds(base, CHUNK)], rows_v, s2)
        c0.wait()
        c1.wait()
        c2.wait()
        d0 = pltpu.async_copy(rows_v, xs_hbm.at[i0_v], s0)
        d1 = pltpu.async_copy(rows_v, xs_hbm.at[i1_v], s1)
        d0.wait()
        d1.wait()

    return k(x, p0, p1)


def _ffn_kernel(bexp_ref, xs_ref, w1_ref, b1_ref, w2_ref, b2_ref, ys_ref):
    del bexp_ref
    xb = xs_ref[...].astype(jnp.bfloat16)
    y = jnp.zeros((BS_G, D), dtype=jnp.float32)
    for ft in range(F // FT):
        w1t = w1_ref[0, :, ft * FT:(ft + 1) * FT]
        b1t = b1_ref[0, 0, ft * FT:(ft + 1) * FT]
        h = jnp.dot(xb, w1t, preferred_element_type=jnp.float32)
        h = h + b1t[None, :]
        h = 0.5 * h * (1.0 + lax.erf(h * 0.7071067811865476))
        w2t = w2_ref[0, ft * FT:(ft + 1) * FT, :]
        y = y + jnp.dot(h.astype(jnp.bfloat16), w2t,
                        preferred_element_type=jnp.float32)
    ys_ref[...] = y + b2_ref[0, 0, :][None, :]


def _grouped_ffn(xs, bexp, W1b, b1, W2b, b2):
    return pl.pallas_call(
        _ffn_kernel,
        grid_spec=pltpu.PrefetchScalarGridSpec(
            num_scalar_prefetch=1,
            grid=(NB,),
            in_specs=[
                pl.BlockSpec((BS_G, D), lambda i, b: (i, 0)),
                pl.BlockSpec((1, D, F), lambda i, b: (b[i], 0, 0)),
                pl.BlockSpec((1, 1, F), lambda i, b: (b[i], 0, 0)),
                pl.BlockSpec((1, F, D), lambda i, b: (b[i], 0, 0)),
                pl.BlockSpec((1, 1, D), lambda i, b: (b[i], 0, 0)),
            ],
            out_specs=pl.BlockSpec((BS_G, D), lambda i, b: (i, 0)),
        ),
        out_shape=jax.ShapeDtypeStruct((P, D), jnp.float32),
        compiler_params=pltpu.CompilerParams(
            dimension_semantics=("parallel",)),
    )(bexp, xs, W1b, b1.reshape(E, 1, F), W2b, b2.reshape(E, 1, D))


def _gather_rows(ys, p0, p1):
    @functools.partial(
        pl.kernel, mesh=_sc_mesh(),
        out_type=[jax.ShapeDtypeStruct((S, D), jnp.float32),
                  jax.ShapeDtypeStruct((S, D), jnp.float32)],
        scratch_types=[
            pltpu.VMEM((CHUNK,), jnp.int32),
            pltpu.VMEM((CHUNK, D), jnp.float32),
        ],
    )
    def k(ys_hbm, p0_hbm, p1_hbm, y0_hbm, y1_hbm, i_v, rows_v):
        wid = lax.axis_index("s") * NC + lax.axis_index("c")
        base = wid * CHUNK
        pltpu.sync_copy(p0_hbm.at[pl.ds(base, CHUNK)], i_v)
        pltpu.sync_copy(ys_hbm.at[i_v], rows_v)
        pltpu.sync_copy(rows_v, y0_hbm.at[pl.ds(base, CHUNK)])
        pltpu.sync_copy(p1_hbm.at[pl.ds(base, CHUNK)], i_v)
        pltpu.sync_copy(ys_hbm.at[i_v], rows_v)
        pltpu.sync_copy(rows_v, y1_hbm.at[pl.ds(base, CHUNK)])

    return k(ys, p0, p1)


def _combine_kernel(y0_ref, y1_ref, w0_ref, w1_ref, out_ref):
    out_ref[...] = w0_ref[...] * y0_ref[...] + w1_ref[...] * y1_ref[...]


def _combine(y0, y1, w0, w1):
    TSC = 512
    return pl.pallas_call(
        _combine_kernel,
        grid=(S // TSC,),
        in_specs=[
            pl.BlockSpec((TSC, D), lambda i: (i, 0)),
            pl.BlockSpec((TSC, D), lambda i: (i, 0)),
            pl.BlockSpec((TSC, 1), lambda i: (i, 0)),
            pl.BlockSpec((TSC, 1), lambda i: (i, 0)),
        ],
        out_specs=pl.BlockSpec((TSC, D), lambda i: (i, 0)),
        out_shape=jax.ShapeDtypeStruct((S, D), jnp.float32),
    )(y0, y1, w0, w1)


@jax.jit
def _moe(x, rw, rb, W1, b1, W2, b2):
    xf = x.reshape(S, D)
    p0, p1, w0, w1, bexp = _router(xf, rw, rb)
    p0f, p1f = p0.reshape(S), p1.reshape(S)
    xs = _scatter_rows(xf, p0f, p1f)
    ys = _grouped_ffn(xs, bexp.reshape(NB),
                      W1.astype(jnp.bfloat16), b1,
                      W2.astype(jnp.bfloat16), b2)
    y0, y1 = _gather_rows(ys, p0f, p1f)
    return _combine(y0, y1, w0, w1)


def kernel(x, rw, rb, W1, b1, W2, b2):
    leading = x.shape[:-1]
    out = _moe(x, rw, rb, W1, b1, W2, b2)
    return out.reshape(*leading, D)
